# baseline (speedup 1.0000x reference)
import functools

import jax
import jax.numpy as jnp
from jax import lax
from jax.experimental import pallas as pl
from jax.experimental.pallas import tpu as pltpu
from jax.experimental.pallas import tpu_sc as plsc

N = 10000
E = 320000
D = 128

NC = 2
NS = 16
NW = NC * NS
E_PER_TILE = E // NW
K = 96
NFULL = E_PER_TILE // K
TAIL = E_PER_TILE - NFULL * K
ROWS_PER_SUBCORE = N // NS


def _sc_segment_sum(hc, ef):
    mesh = plsc.VectorSubcoreMesh(core_axis_name="c", subcore_axis_name="s")

    @functools.partial(
        pl.kernel,
        out_type=jax.ShapeDtypeStruct((NC, N, D), jnp.float32),
        mesh=mesh,
        scratch_types=[
            pltpu.VMEM((6, 2, K), jnp.int32),
            pltpu.VMEM((2, TAIL), jnp.int32),
            pltpu.VMEM((K, D), jnp.float32),
            pltpu.VMEM((K, D), jnp.float32),
            pltpu.VMEM((K, D), jnp.float32),
            pltpu.VMEM_SHARED((N, D), jnp.float32),
        ] + [pltpu.SemaphoreType.DMA] * 13,
    )
    def seg(hc_hbm, ef_hbm, out_hbm,
            idx_v, tidx_v, rows0_v, rows1_v, rows2_v, acc_sh,
            semG0, semG1, semG2, semS0, semS1, semS2, semT,
            si0, si1, si2, si3, si4, si5):
        semi = (si0, si1, si2, si3, si4, si5)
        semG = (semG0, semG1, semG2)
        semS = (semS0, semS1, semS2)
        rows = (rows0_v, rows1_v, rows2_v)
        cid = lax.axis_index("c")
        sid = lax.axis_index("s")
        wid = sid * NC + cid
        ebase = wid * E_PER_TILE

        def load_idx(chunk, slot, sem):
            e0 = ebase + chunk * K
            pltpu.async_copy(ef_hbm.at[pl.ds(e0, K)],
                             idx_v.at[slot, 0], sem)
            pltpu.async_copy(ef_hbm.at[pl.ds(E + e0, K)],
                             idx_v.at[slot, 1], sem)

        def wait_idx(chunk, slot, sem):
            e0 = ebase + chunk * K
            pltpu.make_async_copy(ef_hbm.at[pl.ds(e0, K)],
                                  idx_v.at[slot, 0], sem).wait()
            pltpu.make_async_copy(ef_hbm.at[pl.ds(E + e0, K)],
                                  idx_v.at[slot, 1], sem).wait()

        def gather(slot, rows, sem):
            pltpu.async_copy(hc_hbm.at[idx_v.at[slot, 0]], rows, sem)

        def wait_gather(slot, rows, sem):
            pltpu.make_async_copy(hc_hbm.at[idx_v.at[slot, 0]], rows,
                                  sem).wait()

        def scatter(slot, rows, sem):
            pltpu.async_copy(rows, acc_sh.at[idx_v.at[slot, 1]], sem,
                             add=True)

        def wait_scatter(slot, rows, sem):
            pltpu.make_async_copy(rows, acc_sh.at[idx_v.at[slot, 1]],
                                  sem).wait()

        for c0 in range(5):
            load_idx(c0, c0, semi[c0])

        zero16 = jnp.zeros((16,), jnp.float32)

        @pl.loop(0, K)
        def _(i):
            @pl.loop(0, D, step=16)
            def _(j):
                rows0_v[i, pl.ds(j, 16)] = zero16

        row0 = sid * ROWS_PER_SUBCORE
        nz = ROWS_PER_SUBCORE // K
        rz = ROWS_PER_SUBCORE - nz * K

        for i in range(ROWS_PER_SUBCORE // K):
            pltpu.async_copy(rows0_v, acc_sh.at[pl.ds(row0 + i * K, K)],
                             semG[i % 3])
        pltpu.async_copy(rows0_v.at[pl.ds(0, rz)],
                         acc_sh.at[pl.ds(row0 + nz * K, rz)], semT)
        for i in range(ROWS_PER_SUBCORE // K):
            pltpu.make_async_copy(
                rows0_v, acc_sh.at[pl.ds(row0 + i * K, K)],
                semG[i % 3]).wait()
        pltpu.make_async_copy(rows0_v.at[pl.ds(0, rz)],
                              acc_sh.at[pl.ds(row0 + nz * K, rz)],
                              semT).wait()

        wait_idx(0, 0, semi[0])
        gather(0, rows0_v, semG[0])
        wait_idx(1, 1, semi[1])
        gather(1, rows1_v, semG[1])
        plsc.subcore_barrier()

        @pl.loop(0, NFULL // 6)
        def _(jj):
            j0 = jj * 6
            for u in range(6):
                c = j0 + u
                b = u % 3
                pb = (u + 2) % 3
                s2 = (u + 2) % 6
                sp = (u + 5) % 6
                wait_idx(c + 2, s2, semi[s2])
                wait_gather(u % 6, rows[b], semG[b])
                if u == 0:
                    @pl.when(jj > 0)
                    def _():
                        wait_scatter(sp, rows[pb], semS[pb])
                else:
                    wait_scatter(sp, rows[pb], semS[pb])
                gather(s2, rows[pb], semG[pb])
                @pl.when(c + 5 < NFULL)
                def _():
                    load_idx(c + 5, sp, semi[sp])
                scatter(u % 6, rows[b], semS[b])

        et0 = ebase + NFULL * K
        pltpu.async_copy(ef_hbm.at[pl.ds(et0, TAIL)], tidx_v.at[0], semT)
        pltpu.async_copy(ef_hbm.at[pl.ds(E + et0, TAIL)], tidx_v.at[1],
                         semT)
        wait_gather(0, rows0_v, semG0)
        wait_scatter(5, rows2_v, semS2)
        scatter(0, rows0_v, semS0)
        wait_gather(1, rows1_v, semG1)
        pltpu.make_async_copy(ef_hbm.at[pl.ds(et0, TAIL)], tidx_v.at[0],
                              semT).wait()
        pltpu.make_async_copy(ef_hbm.at[pl.ds(E + et0, TAIL)],
                              tidx_v.at[1], semT).wait()
        pltpu.async_copy(hc_hbm.at[tidx_v.at[0]],
                         rows2_v.at[pl.ds(0, TAIL)], semG2)
        scatter(1, rows1_v, semS1)
        pltpu.make_async_copy(hc_hbm.at[tidx_v.at[0]],
                              rows2_v.at[pl.ds(0, TAIL)], semG2).wait()
        pltpu.sync_copy(rows2_v.at[pl.ds(0, TAIL)],
                        acc_sh.at[tidx_v.at[1]], add=True)
        wait_scatter(0, rows0_v, semS0)
        wait_scatter(1, rows1_v, semS1)

        plsc.subcore_barrier()

        wb = (N // NS) // 8 * 8
        pltpu.sync_copy(acc_sh.at[pl.ds(sid * wb, wb)],
                        out_hbm.at[cid, pl.ds(sid * wb, wb)])

        @pl.when(sid == 0)
        def _():
            pltpu.sync_copy(acc_sh.at[pl.ds(NS * wb, N - NS * wb)],
                            out_hbm.at[cid, pl.ds(NS * wb, N - NS * wb)])

    return seg(hc, ef)


_BN = N // 2
_row_spec = pl.BlockSpec((_BN, D), lambda i: (i, 0))
_p_spec = pl.BlockSpec((NC, _BN, D), lambda i: (0, i, 0))
_w_spec = pl.BlockSpec((D, D), lambda i: (0, 0))
_b_spec = pl.BlockSpec((1, D), lambda i: (0, 0))


def _tc_clip(x, ei):
    def body(x_ref, e_ref, o_ref, f_ref):
        xb = x_ref[...]
        n2 = jnp.sum(xb * xb, axis=1, keepdims=True)
        scale = 1.0 / jnp.maximum(jnp.sqrt(n2), 1.0)
        o_ref[...] = xb * scale
        f_ref[pl.ds(0, E)] = e_ref[0, :]
        f_ref[pl.ds(E, E)] = e_ref[1, :]

    return pl.pallas_call(
        body, out_shape=(jax.ShapeDtypeStruct((N, D), jnp.float32),
                         jax.ShapeDtypeStruct((2 * E,), jnp.int32)))(x, ei)


def _tc_layer0(x, hc, p, W0, b0):
    def body(x_ref, hc_ref, p_ref, w_ref, b_ref, o_ref):
        agg = hc_ref[...] + p_ref[0] + p_ref[1]
        out0 = jnp.dot(agg, w_ref[...],
                       preferred_element_type=jnp.float32,
                       precision=lax.Precision.HIGHEST)
        h = x_ref[...] + jnp.maximum(out0 + b_ref[...], 0.0)
        n2 = jnp.sum(h * h, axis=1, keepdims=True)
        scale = 1.0 / jnp.maximum(jnp.sqrt(n2), 1.0)
        o_ref[...] = h * scale

    return pl.pallas_call(
        body, grid=(2,),
        in_specs=[_row_spec, _row_spec, _p_spec, _w_spec, _b_spec],
        out_specs=_row_spec,
        out_shape=jax.ShapeDtypeStruct((N, D), jnp.float32))(
            x, hc, p, W0, b0.reshape(1, D))


def _tc_layer1(hc, p, W1, b1):
    def body(hc_ref, p_ref, w_ref, b_ref, o_ref):
        agg = hc_ref[...] + p_ref[0] + p_ref[1]
        o_ref[...] = jnp.dot(agg, w_ref[...],
                             preferred_element_type=jnp.float32,
                             precision=lax.Precision.HIGHEST) + b_ref[...]

    return pl.pallas_call(
        body, grid=(2,),
        in_specs=[_row_spec, _p_spec, _w_spec, _b_spec],
        out_specs=_row_spec,
        out_shape=jax.ShapeDtypeStruct((N, D), jnp.float32))(
            hc, p, W1, b1.reshape(1, D))


def kernel(x, edge_index, W0, b0, W1, b1):
    hc0, ef = _tc_clip(x, edge_index.astype(jnp.int32))
    p0 = _sc_segment_sum(hc0, ef)
    hc1 = _tc_layer0(x, hc0, p0, W0, b0)
    p1 = _sc_segment_sum(hc1, ef)
    return _tc_layer1(hc1, p1, W1, b1)


# speedup vs baseline: 1.0247x; 1.0247x over previous
import functools

import jax
import jax.numpy as jnp
from jax import lax
from jax.experimental import pallas as pl
from jax.experimental.pallas import tpu as pltpu
from jax.experimental.pallas import tpu_sc as plsc

N = 10000
E = 320000
D = 128

NC = 2
NS = 16
NW = NC * NS
E_PER_TILE = E // NW
K = 96
NFULL = E_PER_TILE // K
TAIL = E_PER_TILE - NFULL * K
ROWS_PER_SUBCORE = N // NS


def _sc_segment_sum(hc, ef):
    mesh = plsc.VectorSubcoreMesh(core_axis_name="c", subcore_axis_name="s")

    @functools.partial(
        pl.kernel,
        out_type=jax.ShapeDtypeStruct((NC, N, D), jnp.float32),
        mesh=mesh,
        scratch_types=[
            pltpu.VMEM((6, 2, K), jnp.int32),
            pltpu.VMEM((2, TAIL), jnp.int32),
            pltpu.VMEM((K, D), jnp.float32),
            pltpu.VMEM((K, D), jnp.float32),
            pltpu.VMEM((K, D), jnp.float32),
            pltpu.VMEM_SHARED((N, D), jnp.float32),
        ] + [pltpu.SemaphoreType.DMA] * 13,
    )
    def seg(hc_hbm, ef_hbm, out_hbm,
            idx_v, tidx_v, rows0_v, rows1_v, rows2_v, acc_sh,
            semG0, semG1, semG2, semS0, semS1, semS2, semT,
            si0, si1, si2, si3, si4, si5):
        semi = (si0, si1, si2, si3, si4, si5)
        semG = (semG0, semG1, semG2)
        semS = (semS0, semS1, semS2)
        rows = (rows0_v, rows1_v, rows2_v)
        cid = lax.axis_index("c")
        sid = lax.axis_index("s")
        wid = sid * NC + cid
        ebase = wid * E_PER_TILE

        def load_idx(chunk, slot, sem):
            e0 = ebase + chunk * K
            pltpu.async_copy(ef_hbm.at[pl.ds(e0, K)],
                             idx_v.at[slot, 0], sem)
            pltpu.async_copy(ef_hbm.at[pl.ds(E + e0, K)],
                             idx_v.at[slot, 1], sem)

        def wait_idx(chunk, slot, sem):
            e0 = ebase + chunk * K
            pltpu.make_async_copy(ef_hbm.at[pl.ds(e0, K)],
                                  idx_v.at[slot, 0], sem).wait()
            pltpu.make_async_copy(ef_hbm.at[pl.ds(E + e0, K)],
                                  idx_v.at[slot, 1], sem).wait()

        def gather(slot, rows, sem):
            pltpu.async_copy(hc_hbm.at[idx_v.at[slot, 0]], rows, sem)

        def wait_gather(slot, rows, sem):
            pltpu.make_async_copy(hc_hbm.at[idx_v.at[slot, 0]], rows,
                                  sem).wait()

        def scatter(slot, rows, sem):
            pltpu.async_copy(rows, acc_sh.at[idx_v.at[slot, 1]], sem,
                             add=True)

        def wait_scatter(slot, rows, sem):
            pltpu.make_async_copy(rows, acc_sh.at[idx_v.at[slot, 1]],
                                  sem).wait()

        for c0 in range(5):
            load_idx(c0, c0, semi[c0])

        zero16 = jnp.zeros((16,), jnp.float32)

        @pl.loop(0, K)
        def _(i):
            @pl.loop(0, D, step=16)
            def _(j):
                rows0_v[i, pl.ds(j, 16)] = zero16

        row0 = sid * ROWS_PER_SUBCORE
        nz = ROWS_PER_SUBCORE // K
        rz = ROWS_PER_SUBCORE - nz * K

        for i in range(ROWS_PER_SUBCORE // K):
            pltpu.async_copy(rows0_v, acc_sh.at[pl.ds(row0 + i * K, K)],
                             semG[i % 3])
        pltpu.async_copy(rows0_v.at[pl.ds(0, rz)],
                         acc_sh.at[pl.ds(row0 + nz * K, rz)], semT)
        for i in range(ROWS_PER_SUBCORE // K):
            pltpu.make_async_copy(
                rows0_v, acc_sh.at[pl.ds(row0 + i * K, K)],
                semG[i % 3]).wait()
        pltpu.make_async_copy(rows0_v.at[pl.ds(0, rz)],
                              acc_sh.at[pl.ds(row0 + nz * K, rz)],
                              semT).wait()

        wait_idx(0, 0, semi[0])
        gather(0, rows0_v, semG[0])
        wait_idx(1, 1, semi[1])
        gather(1, rows1_v, semG[1])
        plsc.subcore_barrier()

        @pl.loop(0, NFULL // 6)
        def _(jj):
            j0 = jj * 6
            for u in range(6):
                c = j0 + u
                b = u % 3
                pb = (u + 2) % 3
                s2 = (u + 2) % 6
                sp = (u + 5) % 6
                wait_idx(c + 2, s2, semi[s2])
                wait_gather(u % 6, rows[b], semG[b])
                if u == 0:
                    @pl.when(jj > 0)
                    def _():
                        wait_scatter(sp, rows[pb], semS[pb])
                else:
                    wait_scatter(sp, rows[pb], semS[pb])
                gather(s2, rows[pb], semG[pb])
                @pl.when(c + 5 < NFULL)
                def _():
                    load_idx(c + 5, sp, semi[sp])
                scatter(u % 6, rows[b], semS[b])

        et0 = ebase + NFULL * K
        pltpu.async_copy(ef_hbm.at[pl.ds(et0, TAIL)], tidx_v.at[0], semT)
        pltpu.async_copy(ef_hbm.at[pl.ds(E + et0, TAIL)], tidx_v.at[1],
                         semT)
        wait_gather(0, rows0_v, semG0)
        wait_scatter(5, rows2_v, semS2)
        scatter(0, rows0_v, semS0)
        wait_gather(1, rows1_v, semG1)
        pltpu.make_async_copy(ef_hbm.at[pl.ds(et0, TAIL)], tidx_v.at[0],
                              semT).wait()
        pltpu.make_async_copy(ef_hbm.at[pl.ds(E + et0, TAIL)],
                              tidx_v.at[1], semT).wait()
        pltpu.async_copy(hc_hbm.at[tidx_v.at[0]],
                         rows2_v.at[pl.ds(0, TAIL)], semG2)
        scatter(1, rows1_v, semS1)
        pltpu.make_async_copy(hc_hbm.at[tidx_v.at[0]],
                              rows2_v.at[pl.ds(0, TAIL)], semG2).wait()
        pltpu.sync_copy(rows2_v.at[pl.ds(0, TAIL)],
                        acc_sh.at[tidx_v.at[1]], add=True)
        wait_scatter(0, rows0_v, semS0)
        wait_scatter(1, rows1_v, semS1)

        plsc.subcore_barrier()

        wb = (N // NS) // 8 * 8
        pltpu.sync_copy(acc_sh.at[pl.ds(sid * wb, wb)],
                        out_hbm.at[cid, pl.ds(sid * wb, wb)])

        @pl.when(sid == 0)
        def _():
            pltpu.sync_copy(acc_sh.at[pl.ds(NS * wb, N - NS * wb)],
                            out_hbm.at[cid, pl.ds(NS * wb, N - NS * wb)])

    return seg(hc, ef)


def _tc_clip(x, ei):
    def body(x_ref, e_ref, o_ref, f_ref):
        xb = x_ref[...]
        n2 = jnp.sum(xb * xb, axis=1, keepdims=True)
        scale = 1.0 / jnp.maximum(jnp.sqrt(n2), 1.0)
        o_ref[...] = xb * scale
        f_ref[pl.ds(0, E)] = e_ref[0, :]
        f_ref[pl.ds(E, E)] = e_ref[1, :]

    return pl.pallas_call(
        body, out_shape=(jax.ShapeDtypeStruct((N, D), jnp.float32),
                         jax.ShapeDtypeStruct((2 * E,), jnp.int32)))(x, ei)


def _tc_layer0(x, hc, p, W0, b0):
    def body(x_ref, hc_ref, p_ref, w_ref, b_ref, o_ref):
        agg = hc_ref[...] + p_ref[0] + p_ref[1]
        out0 = jnp.dot(agg, w_ref[...],
                       preferred_element_type=jnp.float32,
                       precision=lax.Precision.HIGHEST)
        h = x_ref[...] + jnp.maximum(out0 + b_ref[...], 0.0)
        n2 = jnp.sum(h * h, axis=1, keepdims=True)
        scale = 1.0 / jnp.maximum(jnp.sqrt(n2), 1.0)
        o_ref[...] = h * scale

    return pl.pallas_call(
        body, out_shape=jax.ShapeDtypeStruct((N, D), jnp.float32))(
            x, hc, p, W0, b0.reshape(1, D))


def _tc_layer1(hc, p, W1, b1):
    def body(hc_ref, p_ref, w_ref, b_ref, o_ref):
        agg = hc_ref[...] + p_ref[0] + p_ref[1]
        o_ref[...] = jnp.dot(agg, w_ref[...],
                             preferred_element_type=jnp.float32,
                             precision=lax.Precision.HIGHEST) + b_ref[...]

    return pl.pallas_call(
        body, out_shape=jax.ShapeDtypeStruct((N, D), jnp.float32))(
            hc, p, W1, b1.reshape(1, D))


def kernel(x, edge_index, W0, b0, W1, b1):
    hc0, ef = _tc_clip(x, edge_index.astype(jnp.int32))
    p0 = _sc_segment_sum(hc0, ef)
    hc1 = _tc_layer0(x, hc0, p0, W0, b0)
    p1 = _sc_segment_sum(hc1, ef)
    return _tc_layer1(hc1, p1, W1, b1)


# speedup vs baseline: 1.0776x; 1.0516x over previous
import functools

import jax
import jax.numpy as jnp
from jax import lax
from jax.experimental import pallas as pl
from jax.experimental.pallas import tpu as pltpu
from jax.experimental.pallas import tpu_sc as plsc

N = 10000
E = 320000
D = 128

NC = 2
NS = 16
NW = NC * NS
E_PER_TILE = E // NW
K = 72
NFULL = E_PER_TILE // K
TAIL = E_PER_TILE - NFULL * K
ROWS_PER_SUBCORE = N // NS
NB = 4
NR = 8
UN = 8


def _sc_segment_sum(hc, ef):
    mesh = plsc.VectorSubcoreMesh(core_axis_name="c", subcore_axis_name="s")

    @functools.partial(
        pl.kernel,
        out_type=jax.ShapeDtypeStruct((NC, N, D), jnp.float32),
        mesh=mesh,
        scratch_types=[
            pltpu.VMEM((NR, 2, K), jnp.int32),
            pltpu.VMEM((2, TAIL), jnp.int32),
            pltpu.VMEM((NB, K, D), jnp.float32),
            pltpu.VMEM_SHARED((N, D), jnp.float32),
        ] + [pltpu.SemaphoreType.DMA] * (2 * NB + NR + 1),
    )
    def seg(hc_hbm, ef_hbm, out_hbm, idx_v, tidx_v, rows_v, acc_sh, *sems):
        semG = sems[:NB]
        semS = sems[NB:2 * NB]
        semi = sems[2 * NB:2 * NB + NR]
        semT = sems[2 * NB + NR]
        rows = tuple(rows_v.at[i] for i in range(NB))
        cid = lax.axis_index("c")
        sid = lax.axis_index("s")
        wid = sid * NC + cid
        ebase = wid * E_PER_TILE

        def load_idx(chunk, slot, sem):
            e0 = ebase + chunk * K
            pltpu.async_copy(ef_hbm.at[pl.ds(e0, K)],
                             idx_v.at[slot, 0], sem)
            pltpu.async_copy(ef_hbm.at[pl.ds(E + e0, K)],
                             idx_v.at[slot, 1], sem)

        def wait_idx(chunk, slot, sem):
            e0 = ebase + chunk * K
            pltpu.make_async_copy(ef_hbm.at[pl.ds(e0, K)],
                                  idx_v.at[slot, 0], sem).wait()
            pltpu.make_async_copy(ef_hbm.at[pl.ds(E + e0, K)],
                                  idx_v.at[slot, 1], sem).wait()

        def gather(slot, buf, sem):
            pltpu.async_copy(hc_hbm.at[idx_v.at[slot, 0]], buf, sem)

        def wait_gather(slot, buf, sem):
            pltpu.make_async_copy(hc_hbm.at[idx_v.at[slot, 0]], buf,
                                  sem).wait()

        def scatter(slot, buf, sem):
            pltpu.async_copy(buf, acc_sh.at[idx_v.at[slot, 1]], sem,
                             add=True)

        def wait_scatter(slot, buf, sem):
            pltpu.make_async_copy(buf, acc_sh.at[idx_v.at[slot, 1]],
                                  sem).wait()

        for c0 in range(NR - 1):
            load_idx(c0, c0, semi[c0])

        zbuf = rows[NB - 1]
        zero16 = jnp.zeros((16,), jnp.float32)

        @pl.loop(0, K)
        def _(i):
            @pl.loop(0, D, step=16)
            def _(j):
                zbuf[i, pl.ds(j, 16)] = zero16

        row0 = sid * ROWS_PER_SUBCORE
        nz = ROWS_PER_SUBCORE // K
        rz = ROWS_PER_SUBCORE - nz * K

        for i in range(nz):
            pltpu.async_copy(zbuf, acc_sh.at[pl.ds(row0 + i * K, K)],
                             semS[i % NB])
        pltpu.async_copy(zbuf.at[pl.ds(0, rz)],
                         acc_sh.at[pl.ds(row0 + nz * K, rz)], semT)

        for c0 in range(NB - 1):
            wait_idx(c0, c0, semi[c0])
            gather(c0, rows[c0], semG[c0])

        for i in range(nz):
            pltpu.make_async_copy(zbuf, acc_sh.at[pl.ds(row0 + i * K, K)],
                                  semS[i % NB]).wait()
        pltpu.make_async_copy(zbuf.at[pl.ds(0, rz)],
                              acc_sh.at[pl.ds(row0 + nz * K, rz)],
                              semT).wait()
        plsc.subcore_barrier()

        @pl.loop(0, NFULL // UN)
        def _(jj):
            j0 = jj * UN
            for u in range(UN):
                c = j0 + u
                b = u % NB
                gb = (u + 3) % NB
                s3 = (u + 3) % NR
                sp = (u + NR - 1) % NR
                wait_gather(u % NR, rows[b], semG[b])
                if u == 0:
                    @pl.when(jj > 0)
                    def _():
                        wait_scatter(sp, rows[gb], semS[gb])
                else:
                    wait_scatter(sp, rows[gb], semS[gb])

                @pl.when(c + 3 < NFULL)
                def _():
                    wait_idx(c + 3, s3, semi[s3])
                    gather(s3, rows[gb], semG[gb])

                @pl.when(c + NR - 1 < NFULL)
                def _():
                    load_idx(c + NR - 1, sp, semi[sp])
                scatter(u % NR, rows[b], semS[b])

        et0 = ebase + NFULL * K
        pltpu.async_copy(ef_hbm.at[pl.ds(et0, TAIL)], tidx_v.at[0], semT)
        pltpu.async_copy(ef_hbm.at[pl.ds(E + et0, TAIL)], tidx_v.at[1],
                         semT)
        wait_gather(0, rows[0], semG[0])
        wait_scatter(NR - 1, rows[3], semS[3])
        scatter(0, rows[0], semS[0])
        wait_gather(1, rows[1], semG[1])
        pltpu.make_async_copy(ef_hbm.at[pl.ds(et0, TAIL)], tidx_v.at[0],
                              semT).wait()
        pltpu.make_async_copy(ef_hbm.at[pl.ds(E + et0, TAIL)],
                              tidx_v.at[1], semT).wait()
        pltpu.async_copy(hc_hbm.at[tidx_v.at[0]],
                         rows[2].at[pl.ds(0, TAIL)], semG[2])
        scatter(1, rows[1], semS[1])
        pltpu.make_async_copy(hc_hbm.at[tidx_v.at[0]],
                              rows[2].at[pl.ds(0, TAIL)], semG[2]).wait()
        pltpu.sync_copy(rows[2].at[pl.ds(0, TAIL)],
                        acc_sh.at[tidx_v.at[1]], add=True)
        wait_scatter(0, rows[0], semS[0])
        wait_scatter(1, rows[1], semS[1])

        plsc.subcore_barrier()

        wb = (N // NS) // 8 * 8
        pltpu.sync_copy(acc_sh.at[pl.ds(sid * wb, wb)],
                        out_hbm.at[cid, pl.ds(sid * wb, wb)])

        @pl.when(sid == 0)
        def _():
            pltpu.sync_copy(acc_sh.at[pl.ds(NS * wb, N - NS * wb)],
                            out_hbm.at[cid, pl.ds(NS * wb, N - NS * wb)])

    return seg(hc, ef)


def _tc_clip(x, ei):
    def body(x_ref, e_ref, o_ref, f_ref):
        xb = x_ref[...]
        n2 = jnp.sum(xb * xb, axis=1, keepdims=True)
        scale = 1.0 / jnp.maximum(jnp.sqrt(n2), 1.0)
        o_ref[...] = xb * scale
        f_ref[pl.ds(0, E)] = e_ref[0, :]
        f_ref[pl.ds(E, E)] = e_ref[1, :]

    return pl.pallas_call(
        body, out_shape=(jax.ShapeDtypeStruct((N, D), jnp.float32),
                         jax.ShapeDtypeStruct((2 * E,), jnp.int32)))(x, ei)


def _tc_layer0(x, hc, p, W0, b0):
    def body(x_ref, hc_ref, p_ref, w_ref, b_ref, o_ref):
        agg = hc_ref[...] + p_ref[0] + p_ref[1]
        out0 = jnp.dot(agg, w_ref[...],
                       preferred_element_type=jnp.float32,
                       precision=lax.Precision.HIGHEST)
        h = x_ref[...] + jnp.maximum(out0 + b_ref[...], 0.0)
        n2 = jnp.sum(h * h, axis=1, keepdims=True)
        scale = 1.0 / jnp.maximum(jnp.sqrt(n2), 1.0)
        o_ref[...] = h * scale

    return pl.pallas_call(
        body, out_shape=jax.ShapeDtypeStruct((N, D), jnp.float32))(
            x, hc, p, W0, b0.reshape(1, D))


def _tc_layer1(hc, p, W1, b1):
    def body(hc_ref, p_ref, w_ref, b_ref, o_ref):
        agg = hc_ref[...] + p_ref[0] + p_ref[1]
        o_ref[...] = jnp.dot(agg, w_ref[...],
                             preferred_element_type=jnp.float32,
                             precision=lax.Precision.HIGHEST) + b_ref[...]

    return pl.pallas_call(
        body, out_shape=jax.ShapeDtypeStruct((N, D), jnp.float32))(
            hc, p, W1, b1.reshape(1, D))


def kernel(x, edge_index, W0, b0, W1, b1):
    hc0, ef = _tc_clip(x, edge_index.astype(jnp.int32))
    p0 = _sc_segment_sum(hc0, ef)
    hc1 = _tc_layer0(x, hc0, p0, W0, b0)
    p1 = _sc_segment_sum(hc1, ef)
    return _tc_layer1(hc1, p1, W1, b1)
